# baseline (speedup 1.0000x reference)
import functools

import numpy as np
import jax
import jax.numpy as jnp
from jax import lax
from jax.experimental import pallas as pl
from jax.experimental.pallas import tpu as pltpu
from jax.experimental.pallas import tpu_sc as plsc

HIDDEN = 768
BATCH = 4
SEQ = 8192
EPS = 1e-5
SCALE = float(np.sqrt(np.float64(HIDDEN)))

NC, NS, LANES = 2, 16, 16
NW = NC * NS
POS_PER_W = SEQ // NW
CH = 32
NCHUNK = POS_PER_W // CH
NSLICE = HIDDEN // LANES


def _build_pe():
    position = np.arange(SEQ, dtype=np.float64)[:, None]
    denom = np.power(1000.0, np.arange(0, HIDDEN, 2, dtype=np.float64) / HIDDEN)
    odd = np.cos(position / denom)
    even = np.sin(position / denom)
    return np.stack([odd, even], axis=-1).reshape(SEQ, HIDDEN).astype(np.float32)


_PE = _build_pe()


def _allreduce_sum2(a, b):
    idx = lax.iota(jnp.int32, LANES)
    for sh in (1, 2, 4, 8):
        perm = jnp.bitwise_xor(idx, jnp.full((LANES,), sh, jnp.int32))
        a = a + a[perm]
        b = b + b[perm]
    return a, b


def _rsqrt_vec(v):
    i = lax.bitcast_convert_type(v, jnp.int32)
    i = jnp.full((LANES,), 0x5F3759DF, jnp.int32) - lax.shift_right_logical(
        i, jnp.full((LANES,), 1, jnp.int32))
    y = lax.bitcast_convert_type(i, jnp.float32)
    half = v * 0.5
    for _ in range(3):
        y = y * (1.5 - half * y * y)
    return y


def _sc_body(ids_hbm, table_hbm, pe_hbm, gamma_hbm, beta_hbm, out_hbm,
             idx_v, pe_v, rows_v, gb_v, sem):
    cid = lax.axis_index("c")
    sid = lax.axis_index("s")
    wid = sid * NC + cid
    pos0 = wid * POS_PER_W

    pltpu.sync_copy(ids_hbm.at[:, pl.ds(pos0, POS_PER_W)], idx_v)
    pltpu.sync_copy(gamma_hbm, gb_v.at[0])
    pltpu.sync_copy(beta_hbm, gb_v.at[1])

    def chunk_body(j, carry):
        pos = pos0 + j * CH
        pltpu.sync_copy(pe_hbm.at[pl.ds(pos, CH), :], pe_v)
        for b in range(BATCH):
            pltpu.async_copy(
                table_hbm.at[idx_v.at[b, pl.ds(j * CH, CH)]], rows_v, sem
            ).wait()

            def row_body(r, rcarry):
                sum_v = jnp.zeros((LANES,), jnp.float32)
                sq_v = jnp.zeros((LANES,), jnp.float32)
                for k in range(NSLICE):
                    sl = pl.ds(k * LANES, LANES)
                    x = rows_v[r, sl] * SCALE + pe_v[r, sl]
                    rows_v[r, sl] = x
                    sum_v = sum_v + x
                    sq_v = sq_v + x * x
                sum_v, sq_v = _allreduce_sum2(sum_v, sq_v)
                m = sum_v * (1.0 / HIDDEN)
                ex2 = sq_v * (1.0 / HIDDEN)
                var = ex2 - m * m
                istd = _rsqrt_vec(var + EPS)
                for k in range(NSLICE):
                    sl = pl.ds(k * LANES, LANES)
                    rows_v[r, sl] = ((rows_v[r, sl] - m) * istd * gb_v[0, sl]
                                     + gb_v[1, sl])
                return rcarry

            lax.fori_loop(0, CH, row_body, 0)
            pltpu.sync_copy(rows_v, out_hbm.at[b, pl.ds(pos, CH), :])
        return carry

    lax.fori_loop(0, NCHUNK, chunk_body, 0)


_sc_kernel = pl.kernel(
    _sc_body,
    out_type=jax.ShapeDtypeStruct((BATCH, SEQ, HIDDEN), jnp.float32),
    mesh=plsc.VectorSubcoreMesh(
        core_axis_name="c", subcore_axis_name="s",
        num_cores=NC, num_subcores=NS),
    scratch_types=[
        pltpu.VMEM((BATCH, POS_PER_W), jnp.int32),
        pltpu.VMEM((CH, HIDDEN), jnp.float32),
        pltpu.VMEM((CH, HIDDEN), jnp.float32),
        pltpu.VMEM((2, HIDDEN), jnp.float32),
        pltpu.SemaphoreType.DMA,
    ],
)


def kernel(input_ids, table, gamma, beta):
    pe = jnp.asarray(_PE)
    return _sc_kernel(input_ids, table, pe, gamma, beta)
